# Initial kernel scaffold; baseline (speedup 1.0000x reference)
import jax
import jax.numpy as jnp
from jax.experimental import pallas as pl


def kernel(x, edge_index, edge_attr, W_in, b_in, W1, b1, W2, b2, W3, b3, W_root, b_root, W_ih, W_hh, b_ih, b_hh):
    raise NotImplementedError("write your pallas kernel here")


# speedup vs baseline: 1.0855x; 1.0855x over previous
import functools

import jax
import jax.numpy as jnp
from jax import lax
from jax.experimental import pallas as pl
from jax.experimental.pallas import tpu as pltpu
from jax.experimental.pallas import tpu_sc as plsc

N = 10000
E = 160000
D_IN = 128
D_EDGE = 16
H = 32
L = 2

NC = 2
NS = 16
NW = NC * NS
CHUNK = 128
E_PAD = 163840
NCH = E_PAD // CHUNK
CPW = NCH // NW
WAVE = 8
NWAVES = CPW // WAVE
NPAD = 10240
TRASH = N
ZROWS = NPAD // NS
OROWS = 624

f32 = jnp.float32


def _pre_h_body(x_ref, w_ref, b_ref, o_ref):
    o_ref[...] = (
        jnp.dot(x_ref[...].astype(jnp.bfloat16), w_ref[...],
                preferred_element_type=f32) + b_ref[...]
    )


def _pre_r_body(ea_ref, w1_ref, b1_ref, w2_ref, b2_ref, o_ref):
    t = jnp.dot(ea_ref[...].astype(jnp.bfloat16), w1_ref[...],
                preferred_element_type=f32) + b1_ref[...]
    t = jnp.maximum(t, 0.0)
    t = jnp.dot(t.astype(jnp.bfloat16), w2_ref[...],
                preferred_element_type=f32) + b2_ref[...]
    o_ref[...] = jnp.maximum(t, 0.0)


def _msg_body(hs_ref, r_ref, w3_ref, b3_ref, o_ref):
    ew = jnp.dot(r_ref[...].astype(jnp.bfloat16), w3_ref[...],
                 preferred_element_type=f32) + b3_ref[...]
    ewb = ew.astype(jnp.bfloat16)
    hsf = hs_ref[...].astype(jnp.bfloat16).astype(f32)
    acc = hsf[:, 0:1] * ewb[:, 0:H].astype(f32)
    for hh in range(1, H):
        acc = acc + hsf[:, hh : hh + 1] * ewb[:, hh * H : (hh + 1) * H].astype(f32)
    o_ref[...] = acc


def _gru_body(ha_ref, hb_ref, h_ref, wr_ref, br_ref, wih_ref, whh_ref,
              bih_ref, bhh_ref, o_ref):
    h = h_ref[...]
    hb16 = h.astype(jnp.bfloat16)
    m = (ha_ref[...] + hb_ref[...]
         + jnp.dot(hb16, wr_ref[...], preferred_element_type=f32) + br_ref[...])
    gi = jnp.dot(m.astype(jnp.bfloat16), wih_ref[...],
                 preferred_element_type=f32) + bih_ref[...]
    gh = jnp.dot(hb16, whh_ref[...], preferred_element_type=f32) + bhh_ref[...]
    rg = jax.nn.sigmoid(gi[:, 0:H] + gh[:, 0:H])
    z = jax.nn.sigmoid(gi[:, H : 2 * H] + gh[:, H : 2 * H])
    n = jnp.tanh(gi[:, 2 * H : 3 * H] + rg * gh[:, 2 * H : 3 * H])
    o_ref[...] = (1.0 - z) * n + z * h


def _full(shape):
    return pl.BlockSpec(shape, lambda i: (0,) * len(shape))


def _rows(bs, width):
    return pl.BlockSpec((bs, width), lambda i: (i, 0))


def _pre_h(x, w_in, b_in):
    bs = 2000
    return pl.pallas_call(
        _pre_h_body,
        grid=(N // bs,),
        in_specs=[_rows(bs, D_IN), _full((D_IN, H)), _full((1, H))],
        out_specs=_rows(bs, H),
        out_shape=jax.ShapeDtypeStruct((N, H), f32),
    )(x, w_in, b_in)


def _pre_r(ea, w1, b1, w2, b2):
    bs = 3200
    return pl.pallas_call(
        _pre_r_body,
        grid=(E // bs,),
        in_specs=[_rows(bs, D_EDGE), _full((D_EDGE, H)), _full((1, H)),
                  _full((H, H)), _full((1, H))],
        out_specs=_rows(bs, H),
        out_shape=jax.ShapeDtypeStruct((E, H), f32),
    )(ea, w1, b1, w2, b2)


def _msg(hs, r, w3b, b3row):
    bs = 2048
    return pl.pallas_call(
        _msg_body,
        grid=(E_PAD // bs,),
        in_specs=[_rows(bs, H), _rows(bs, H), _full((H, H * H)),
                  _full((1, H * H))],
        out_specs=_rows(bs, H),
        out_shape=jax.ShapeDtypeStruct((E_PAD, H), f32),
    )(hs, r, w3b, b3row)


def _gru(agg_a, agg_b, h, w_root_l, b_root_l, w_iht, w_hht, b_ih2, b_hh2):
    bs = 2000
    return pl.pallas_call(
        _gru_body,
        grid=(N // bs,),
        in_specs=[_rows(bs, H), _rows(bs, H), _rows(bs, H),
                  _full((H, H)), _full((1, H)),
                  _full((H, 3 * H)), _full((H, 3 * H)),
                  _full((1, 3 * H)), _full((1, 3 * H))],
        out_specs=_rows(bs, H),
        out_shape=jax.ShapeDtypeStruct((N, H), f32),
    )(agg_a, agg_b, h, w_root_l, b_root_l, w_iht, w_hht, b_ih2, b_hh2)


def _sc_mesh():
    return plsc.VectorSubcoreMesh(
        core_axis_name="c", subcore_axis_name="s", num_cores=NC,
        num_subcores=NS)


def _gather(h, src2d):
    @functools.partial(
        pl.kernel,
        out_type=jax.ShapeDtypeStruct((E_PAD, H), f32),
        mesh=_sc_mesh(),
        scratch_types=[
            pltpu.VMEM((CPW, CHUNK), jnp.int32),
            pltpu.VMEM((WAVE, CHUNK, H), f32),
            pltpu.SemaphoreType.DMA,
            pltpu.SemaphoreType.DMA,
        ],
        compiler_params=pltpu.CompilerParams(use_tc_tiling_on_sc=False),
    )
    def k(h_hbm, src_hbm, out_hbm, idx_v, rows_v, gsem, wsem):
        wid = lax.axis_index("s") * NC + lax.axis_index("c")
        pltpu.sync_copy(src_hbm.at[pl.ds(wid * CPW, CPW)], idx_v)
        for wv in range(NWAVES):
            gs = []
            for t in range(WAVE):
                j = wv * WAVE + t
                gs.append(pltpu.async_copy(
                    h_hbm.at[idx_v.at[j]], rows_v.at[t], gsem))
            for d in gs:
                d.wait()
            ws = []
            for t in range(WAVE):
                j = wv * WAVE + t
                ws.append(pltpu.async_copy(
                    rows_v.at[t],
                    out_hbm.at[pl.ds((wid * CPW + j) * CHUNK, CHUNK)], wsem))
            for d in ws:
                d.wait()

    return k(h, src2d)


def _scatter(msg, dst2d, zeros):
    @functools.partial(
        pl.kernel,
        out_type=jax.ShapeDtypeStruct((NC, N, H), f32),
        mesh=_sc_mesh(),
        scratch_types=[
            pltpu.VMEM((CPW, CHUNK), jnp.int32),
            pltpu.VMEM((WAVE, CHUNK, H), f32),
            pltpu.VMEM_SHARED((NPAD, H), f32),
            pltpu.SemaphoreType.DMA,
        ],
        compiler_params=pltpu.CompilerParams(use_tc_tiling_on_sc=False),
    )
    def k(msg_hbm, dst_hbm, zeros_hbm, out_hbm, idx_v, mbuf, agg_sh, lsem):
        cid = lax.axis_index("c")
        sid = lax.axis_index("s")
        wid = sid * NC + cid
        pltpu.sync_copy(zeros_hbm, agg_sh.at[pl.ds(sid * ZROWS, ZROWS)])
        pltpu.sync_copy(dst_hbm.at[pl.ds(wid * CPW, CPW)], idx_v)
        plsc.subcore_barrier()
        for wv in range(NWAVES):
            ls = []
            for t in range(WAVE):
                j = wv * WAVE + t
                ls.append(pltpu.async_copy(
                    msg_hbm.at[pl.ds((wid * CPW + j) * CHUNK, CHUNK)],
                    mbuf.at[t], lsem))
            for d in ls:
                d.wait()
            for t in range(WAVE):
                j = wv * WAVE + t
                pltpu.sync_copy(mbuf.at[t], agg_sh.at[idx_v.at[j]], add=True)
        plsc.subcore_barrier()

        @pl.when(sid < NS - 1)
        def _():
            pltpu.sync_copy(agg_sh.at[pl.ds(sid * OROWS, OROWS)],
                            out_hbm.at[cid, pl.ds(sid * OROWS, OROWS)])

        @pl.when(sid == NS - 1)
        def _():
            last = N - (NS - 1) * OROWS
            pltpu.sync_copy(agg_sh.at[pl.ds((NS - 1) * OROWS, last)],
                            out_hbm.at[cid, pl.ds((NS - 1) * OROWS, last)])

    return k(msg, dst2d, zeros)


def kernel(x, edge_index, edge_attr, W_in, b_in, W1, b1, W2, b2, W3, b3,
           W_root, b_root, W_ih, W_hh, b_ih, b_hh):
    npad = E_PAD - E
    src2d = jnp.concatenate(
        [edge_index[0], jnp.zeros((npad,), jnp.int32)]).reshape(NCH, CHUNK)
    dst2d = jnp.concatenate(
        [edge_index[1], jnp.full((npad,), TRASH, jnp.int32)]).reshape(NCH, CHUNK)

    w3b = W3.astype(jnp.bfloat16)
    b3row = b3.reshape(1, H * H)

    zeros = jnp.zeros((ZROWS, H), f32)

    h = _pre_h(x, W_in.astype(jnp.bfloat16), b_in.reshape(1, H))
    r = _pre_r(edge_attr, W1.astype(jnp.bfloat16), b1.reshape(1, H),
               W2.astype(jnp.bfloat16), b2.reshape(1, H))
    r = jnp.concatenate([r, jnp.zeros((npad, H), f32)], axis=0)

    w_iht = W_ih.T.astype(jnp.bfloat16)
    w_hht = W_hh.T.astype(jnp.bfloat16)
    b_ih2 = b_ih.reshape(1, 3 * H)
    b_hh2 = b_hh.reshape(1, 3 * H)

    for l in range(L):
        hs = _gather(h, src2d)
        msg = _msg(hs, r, w3b, b3row)
        agg2 = _scatter(msg, dst2d, zeros)
        h = _gru(agg2[0], agg2[1], h, W_root[l].astype(jnp.bfloat16),
                 b_root[l].reshape(1, H), w_iht, w_hht, b_ih2, b_hh2)
    return h


# speedup vs baseline: 2.6577x; 2.4483x over previous
import functools

import jax
import jax.numpy as jnp
from jax import lax
from jax.experimental import pallas as pl
from jax.experimental.pallas import tpu as pltpu
from jax.experimental.pallas import tpu_sc as plsc

N = 10000
E = 160000
D_IN = 128
D_EDGE = 16
H = 32
L = 2

NC = 2
NS = 16
NW = NC * NS
CHUNK = 128
E_PAD = 163840
NCH = E_PAD // CHUNK
CPW = NCH // NW
WAVE = 8
NWAVES = CPW // WAVE
NPAD = 10240
TRASH = N
ZROWS = NPAD // NS
OROWS = 624

f32 = jnp.float32


def _pre_h_body(x_ref, w_ref, b_ref, o_ref):
    o_ref[...] = (
        jnp.dot(x_ref[...].astype(jnp.bfloat16), w_ref[...],
                preferred_element_type=f32) + b_ref[...]
    )


def _pre_r_body(ea_ref, w1_ref, b1_ref, w2_ref, b2_ref, o_ref):
    t = jnp.dot(ea_ref[...].astype(jnp.bfloat16), w1_ref[...],
                preferred_element_type=f32) + b1_ref[...]
    t = jnp.maximum(t, 0.0)
    t = jnp.dot(t.astype(jnp.bfloat16), w2_ref[...],
                preferred_element_type=f32) + b2_ref[...]
    o_ref[...] = jnp.maximum(t, 0.0)


def _msg_body(hs_ref, r_ref, w3_ref, b3_ref, rep_ref, o_ref):
    ew = jnp.dot(r_ref[...].astype(jnp.bfloat16), w3_ref[...],
                 preferred_element_type=f32) + b3_ref[...]
    ewf = ew.astype(jnp.bfloat16).astype(f32)
    hsrep = jnp.dot(hs_ref[...].astype(jnp.bfloat16), rep_ref[...],
                    preferred_element_type=f32)
    prod = hsrep * ewf
    t = prod[:, 0:128]
    for g in range(1, (H * H) // 128):
        t = t + prod[:, g * 128 : (g + 1) * 128]
    o_ref[...] = (t[:, 0:H] + t[:, H : 2 * H] + t[:, 2 * H : 3 * H]
                  + t[:, 3 * H : 4 * H])


def _gru_body(ha_ref, hb_ref, h_ref, wr_ref, br_ref, wih_ref, whh_ref,
              bih_ref, bhh_ref, o_ref):
    h = h_ref[...]
    hb16 = h.astype(jnp.bfloat16)
    m = (ha_ref[...] + hb_ref[...]
         + jnp.dot(hb16, wr_ref[...], preferred_element_type=f32) + br_ref[...])
    gi = jnp.dot(m.astype(jnp.bfloat16), wih_ref[...],
                 preferred_element_type=f32) + bih_ref[...]
    gh = jnp.dot(hb16, whh_ref[...], preferred_element_type=f32) + bhh_ref[...]
    rg = jax.nn.sigmoid(gi[:, 0:H] + gh[:, 0:H])
    z = jax.nn.sigmoid(gi[:, H : 2 * H] + gh[:, H : 2 * H])
    n = jnp.tanh(gi[:, 2 * H : 3 * H] + rg * gh[:, 2 * H : 3 * H])
    o_ref[...] = (1.0 - z) * n + z * h


def _full(shape):
    return pl.BlockSpec(shape, lambda i: (0,) * len(shape))


def _rows(bs, width):
    return pl.BlockSpec((bs, width), lambda i: (i, 0))


def _pre_h(x, w_in, b_in):
    bs = 2000
    return pl.pallas_call(
        _pre_h_body,
        grid=(N // bs,),
        in_specs=[_rows(bs, D_IN), _full((D_IN, H)), _full((1, H))],
        out_specs=_rows(bs, H),
        out_shape=jax.ShapeDtypeStruct((N, H), f32),
    )(x, w_in, b_in)


def _pre_r(ea, w1, b1, w2, b2):
    bs = 3200
    return pl.pallas_call(
        _pre_r_body,
        grid=(E // bs,),
        in_specs=[_rows(bs, D_EDGE), _full((D_EDGE, H)), _full((1, H)),
                  _full((H, H)), _full((1, H))],
        out_specs=_rows(bs, H),
        out_shape=jax.ShapeDtypeStruct((E, H), f32),
    )(ea, w1, b1, w2, b2)


def _msg(hs, r, w3b, b3row, rep):
    bs = 2048
    return pl.pallas_call(
        _msg_body,
        grid=(E_PAD // bs,),
        in_specs=[_rows(bs, H), _rows(bs, H), _full((H, H * H)),
                  _full((1, H * H)), _full((H, H * H))],
        out_specs=_rows(bs, H),
        out_shape=jax.ShapeDtypeStruct((E_PAD, H), f32),
    )(hs, r, w3b, b3row, rep)


def _gru(agg_a, agg_b, h, w_root_l, b_root_l, w_iht, w_hht, b_ih2, b_hh2):
    bs = 2000
    return pl.pallas_call(
        _gru_body,
        grid=(N // bs,),
        in_specs=[_rows(bs, H), _rows(bs, H), _rows(bs, H),
                  _full((H, H)), _full((1, H)),
                  _full((H, 3 * H)), _full((H, 3 * H)),
                  _full((1, 3 * H)), _full((1, 3 * H))],
        out_specs=_rows(bs, H),
        out_shape=jax.ShapeDtypeStruct((N, H), f32),
    )(agg_a, agg_b, h, w_root_l, b_root_l, w_iht, w_hht, b_ih2, b_hh2)


def _sc_mesh():
    return plsc.VectorSubcoreMesh(
        core_axis_name="c", subcore_axis_name="s", num_cores=NC,
        num_subcores=NS)


def _gather(h, src2d):
    @functools.partial(
        pl.kernel,
        out_type=jax.ShapeDtypeStruct((E_PAD, H), f32),
        mesh=_sc_mesh(),
        scratch_types=[
            pltpu.VMEM((CPW, CHUNK), jnp.int32),
            pltpu.VMEM((WAVE, CHUNK, H), f32),
            pltpu.SemaphoreType.DMA,
            pltpu.SemaphoreType.DMA,
        ],
        compiler_params=pltpu.CompilerParams(use_tc_tiling_on_sc=False),
    )
    def k(h_hbm, src_hbm, out_hbm, idx_v, rows_v, gsem, wsem):
        wid = lax.axis_index("s") * NC + lax.axis_index("c")
        pltpu.sync_copy(src_hbm.at[pl.ds(wid * CPW, CPW)], idx_v)
        for wv in range(NWAVES):
            gs = []
            for t in range(WAVE):
                j = wv * WAVE + t
                gs.append(pltpu.async_copy(
                    h_hbm.at[idx_v.at[j]], rows_v.at[t], gsem))
            for d in gs:
                d.wait()
            ws = []
            for t in range(WAVE):
                j = wv * WAVE + t
                ws.append(pltpu.async_copy(
                    rows_v.at[t],
                    out_hbm.at[pl.ds((wid * CPW + j) * CHUNK, CHUNK)], wsem))
            for d in ws:
                d.wait()

    return k(h, src2d)


def _scatter(msg, dst2d, zeros):
    @functools.partial(
        pl.kernel,
        out_type=jax.ShapeDtypeStruct((NC, N, H), f32),
        mesh=_sc_mesh(),
        scratch_types=[
            pltpu.VMEM((CPW, CHUNK), jnp.int32),
            pltpu.VMEM((WAVE, CHUNK, H), f32),
            pltpu.VMEM_SHARED((NPAD, H), f32),
            pltpu.SemaphoreType.DMA,
        ],
        compiler_params=pltpu.CompilerParams(use_tc_tiling_on_sc=False),
    )
    def k(msg_hbm, dst_hbm, zeros_hbm, out_hbm, idx_v, mbuf, agg_sh, lsem):
        cid = lax.axis_index("c")
        sid = lax.axis_index("s")
        wid = sid * NC + cid
        pltpu.sync_copy(zeros_hbm, agg_sh.at[pl.ds(sid * ZROWS, ZROWS)])
        pltpu.sync_copy(dst_hbm.at[pl.ds(wid * CPW, CPW)], idx_v)
        plsc.subcore_barrier()
        for wv in range(NWAVES):
            ls = []
            for t in range(WAVE):
                j = wv * WAVE + t
                ls.append(pltpu.async_copy(
                    msg_hbm.at[pl.ds((wid * CPW + j) * CHUNK, CHUNK)],
                    mbuf.at[t], lsem))
            for d in ls:
                d.wait()
            for t in range(WAVE):
                j = wv * WAVE + t
                pltpu.sync_copy(mbuf.at[t], agg_sh.at[idx_v.at[j]], add=True)
        plsc.subcore_barrier()

        @pl.when(sid < NS - 1)
        def _():
            pltpu.sync_copy(agg_sh.at[pl.ds(sid * OROWS, OROWS)],
                            out_hbm.at[cid, pl.ds(sid * OROWS, OROWS)])

        @pl.when(sid == NS - 1)
        def _():
            last = N - (NS - 1) * OROWS
            pltpu.sync_copy(agg_sh.at[pl.ds((NS - 1) * OROWS, last)],
                            out_hbm.at[cid, pl.ds((NS - 1) * OROWS, last)])

    return k(msg, dst2d, zeros)


def kernel(x, edge_index, edge_attr, W_in, b_in, W1, b1, W2, b2, W3, b3,
           W_root, b_root, W_ih, W_hh, b_ih, b_hh):
    npad = E_PAD - E
    src2d = jnp.concatenate(
        [edge_index[0], jnp.zeros((npad,), jnp.int32)]).reshape(NCH, CHUNK)
    dst2d = jnp.concatenate(
        [edge_index[1], jnp.full((npad,), TRASH, jnp.int32)]).reshape(NCH, CHUNK)

    w3b = W3.astype(jnp.bfloat16)
    b3row = b3.reshape(1, H * H)
    rep = jnp.repeat(jnp.eye(H, dtype=jnp.bfloat16), H, axis=1)

    zeros = jnp.zeros((ZROWS, H), f32)

    h = _pre_h(x, W_in.astype(jnp.bfloat16), b_in.reshape(1, H))
    r = _pre_r(edge_attr, W1.astype(jnp.bfloat16), b1.reshape(1, H),
               W2.astype(jnp.bfloat16), b2.reshape(1, H))
    r = jnp.concatenate([r, jnp.zeros((npad, H), f32)], axis=0)

    w_iht = W_ih.T.astype(jnp.bfloat16)
    w_hht = W_hh.T.astype(jnp.bfloat16)
    b_ih2 = b_ih.reshape(1, 3 * H)
    b_hh2 = b_hh.reshape(1, 3 * H)

    for l in range(L):
        hs = _gather(h, src2d)
        msg = _msg(hs, r, w3b, b3row, rep)
        agg2 = _scatter(msg, dst2d, zeros)
        h = _gru(agg2[0], agg2[1], h, W_root[l].astype(jnp.bfloat16),
                 b_root[l].reshape(1, H), w_iht, w_hht, b_ih2, b_hh2)
    return h
